# speedup vs baseline: 3.1097x; 3.1097x over previous
import jax
import jax.numpy as jnp
from jax.experimental import pallas as pl
from jax.experimental.pallas import tpu as pltpu
from jax.experimental.pallas import tpu_sc as plsc

_WINDOW = 128


def kernel(word_ids, table):
    batch, seq = word_ids.shape
    _, embed = table.shape
    num_idx = batch * seq

    idx = word_ids.reshape(1, num_idx).astype(jnp.int32)

    mesh = plsc.VectorSubcoreMesh(
        core_axis_name="core", subcore_axis_name="subcore"
    )

    @pl.kernel(
        out_type=jax.ShapeDtypeStruct((num_idx, embed), table.dtype),
        mesh=mesh,
    )
    def _gather(tab_hbm, idx_hbm, out_hbm):
        def body(i_vmem, o_vmem):
            pltpu.sync_copy(tab_hbm.at[i_vmem.at[0]], o_vmem)

        pltpu.emit_pipeline(
            body,
            grid=(num_idx // _WINDOW,),
            in_specs=[
                pl.BlockSpec((1, _WINDOW), index_map=lambda i: (0, i))
            ],
            out_specs=[
                pl.BlockSpec((_WINDOW, embed), index_map=lambda i: (i, 0))
            ],
            core_axis_name=("core", "subcore"),
            dimension_semantics=(pltpu.PARALLEL,),
        )(idx_hbm, out_hbm)

    out = _gather(table, idx)
    return out.reshape(batch, seq, embed)


# speedup vs baseline: 3.2954x; 1.0597x over previous
import jax
import jax.numpy as jnp
from jax.experimental import pallas as pl
from jax.experimental.pallas import tpu as pltpu
from jax.experimental.pallas import tpu_sc as plsc

_WINDOW = 256


def kernel(word_ids, table):
    batch, seq = word_ids.shape
    _, embed = table.shape
    num_idx = batch * seq

    idx = word_ids.reshape(1, num_idx).astype(jnp.int32)

    mesh = plsc.VectorSubcoreMesh(
        core_axis_name="core", subcore_axis_name="subcore"
    )

    @pl.kernel(
        out_type=jax.ShapeDtypeStruct((num_idx, embed), table.dtype),
        mesh=mesh,
    )
    def _gather(tab_hbm, idx_hbm, out_hbm):
        def body(i_vmem, o_vmem):
            pltpu.sync_copy(tab_hbm.at[i_vmem.at[0]], o_vmem)

        pltpu.emit_pipeline(
            body,
            grid=(num_idx // _WINDOW,),
            in_specs=[
                pl.BlockSpec((1, _WINDOW), index_map=lambda i: (0, i))
            ],
            out_specs=[
                pl.BlockSpec((_WINDOW, embed), index_map=lambda i: (i, 0))
            ],
            core_axis_name=("core", "subcore"),
            dimension_semantics=(pltpu.PARALLEL,),
        )(idx_hbm, out_hbm)

    out = _gather(table, idx)
    return out.reshape(batch, seq, embed)


# speedup vs baseline: 4.2330x; 1.2845x over previous
import jax
import jax.numpy as jnp
from jax.experimental import pallas as pl
from jax.experimental.pallas import tpu as pltpu
from jax.experimental.pallas import tpu_sc as plsc

_BBLK = 8


def kernel(word_ids, table):
    batch, seq = word_ids.shape
    _, embed = table.shape
    window = _BBLK * seq

    idx = word_ids.reshape(batch // _BBLK, _BBLK, seq).astype(jnp.int32)

    mesh = plsc.VectorSubcoreMesh(
        core_axis_name="core", subcore_axis_name="subcore"
    )

    @pl.kernel(
        out_type=jax.ShapeDtypeStruct((batch, seq, embed), table.dtype),
        mesh=mesh,
    )
    def _gather(tab_hbm, idx_hbm, out_hbm):
        def body(i_vmem, o_vmem):
            @pl.loop(0, _BBLK)
            def _(j):
                pltpu.sync_copy(
                    tab_hbm.at[i_vmem.at[0, j]],
                    o_vmem.at[j],
                )

        pltpu.emit_pipeline(
            body,
            grid=(batch // _BBLK,),
            in_specs=[
                pl.BlockSpec((1, _BBLK, seq), index_map=lambda i: (i, 0, 0))
            ],
            out_specs=[
                pl.BlockSpec(
                    (_BBLK, seq, embed), index_map=lambda i: (i, 0, 0)
                )
            ],
            core_axis_name=("core", "subcore"),
            dimension_semantics=(pltpu.PARALLEL,),
        )(idx_hbm, out_hbm)

    return _gather(table, idx)


# speedup vs baseline: 5.9038x; 1.3947x over previous
import jax
import jax.numpy as jnp
from jax.experimental import pallas as pl
from jax.experimental.pallas import tpu as pltpu
from jax.experimental.pallas import tpu_sc as plsc

_BBLK = 8


def kernel(word_ids, table):
    batch, seq = word_ids.shape
    _, embed = table.shape

    idx = word_ids.astype(jnp.int32)

    mesh = plsc.VectorSubcoreMesh(
        core_axis_name="core", subcore_axis_name="subcore"
    )

    @pl.kernel(
        out_type=jax.ShapeDtypeStruct((batch, seq, embed), table.dtype),
        mesh=mesh,
        scratch_types=[pltpu.SemaphoreType.DMA],
    )
    def _gather(tab_hbm, idx_hbm, out_hbm, sem):
        def body(i_vmem, o_vmem):
            copies = [
                pltpu.async_copy(
                    tab_hbm.at[i_vmem.at[j]], o_vmem.at[j], sem
                )
                for j in range(_BBLK)
            ]
            for c in copies:
                c.wait()

        pltpu.emit_pipeline(
            body,
            grid=(batch // _BBLK,),
            in_specs=[
                pl.BlockSpec((_BBLK, seq), index_map=lambda i: (i, 0))
            ],
            out_specs=[
                pl.BlockSpec(
                    (_BBLK, seq, embed), index_map=lambda i: (i, 0, 0)
                )
            ],
            core_axis_name=("core", "subcore"),
            dimension_semantics=(pltpu.PARALLEL,),
        )(idx_hbm, out_hbm)

    return _gather(table, idx)
